# baseline (speedup 1.0000x reference)
import functools

import jax
import jax.numpy as jnp
from jax import lax
from jax.experimental import pallas as pl
from jax.experimental import layout as jexp_layout
from jax.experimental.pallas import tpu as pltpu
from jax.experimental.pallas import tpu_sc as plsc

HIDDEN = 64
PAIR = 64
TCF_BLK = 4096
N_TABLE = 1000000
T0_BLK = 8192


def _t0_body(t_ref, o_ref):
    o_ref[...] = jnp.maximum(t_ref[...].T, 0.0)


def _make_gather(B, n_workers, b_per_w, n_chunks):
    mesh = plsc.VectorSubcoreMesh(core_axis_name="c", subcore_axis_name="s")
    n_pairs = n_chunks // 2
    pairs_per_w = b_per_w // 2

    @functools.partial(
        pl.kernel,
        mesh=mesh,
        out_type=jax.ShapeDtypeStruct((B // 2, 2 * HIDDEN), jnp.float32),
        scratch_types=[
            pltpu.VMEM((n_chunks, 2 * PAIR), jnp.int32),
            pltpu.VMEM((2, 2, PAIR, HIDDEN), jnp.float32),
            pltpu.SemaphoreType.DMA,
            pltpu.SemaphoreType.DMA,
        ],
        compiler_params=pltpu.CompilerParams(use_tc_tiling_on_sc=False),
    )
    def gather_k(idx_hbm, table_hbm, out_hbm, idx_v, rows_v, sem0, sem1):
        nc = lax.axis_size("c")
        wid = lax.axis_index("s") * nc + lax.axis_index("c")
        pair_base = wid * pairs_per_w

        pltpu.sync_copy(idx_hbm.at[wid], idx_v)

        def copies(c, buf, sem):
            for g in range(2):
                src = table_hbm.at[idx_v.at[c, pl.ds(PAIR * g, PAIR)]]
                dst = rows_v.at[buf, g]
                yield src, dst, sem

        def start(c, buf, sem):
            for src, dst, s in copies(c, buf, sem):
                pltpu.async_copy(src, dst, s)

        def wait(c, buf, sem):
            for src, dst, s in copies(c, buf, sem):
                pltpu.make_async_copy(src, dst, s).wait()

        def store(c, buf):
            for g in range(2):
                pltpu.sync_copy(
                    rows_v.at[buf, g],
                    out_hbm.at[pl.ds(pair_base + c * PAIR, PAIR),
                               pl.ds(HIDDEN * g, HIDDEN)])

        start(0, 0, sem0)

        def body(i, carry):
            c0 = 2 * i
            start(c0 + 1, 1, sem1)
            wait(c0, 0, sem0)
            store(c0, 0)
            start(c0 + 2, 0, sem0)
            wait(c0 + 1, 1, sem1)
            store(c0 + 1, 1)
            return carry

        lax.fori_loop(0, n_pairs - 1, body, 0)

        c0 = n_chunks - 2
        start(c0 + 1, 1, sem1)
        wait(c0, 0, sem0)
        store(c0, 0)
        wait(c0 + 1, 1, sem1)
        store(c0 + 1, 1)

    return gather_k


def _tcf_body(x_ref, wd_ref, b_ref, o_ref):
    y = lax.dot_general(
        x_ref[...], wd_ref[...], (((1,), (0,)), ((), ())),
        preferred_element_type=jnp.float32) + b_ref[...]
    o_ref[0] = y[:, :HIDDEN]
    o_ref[1] = y[:, HIDDEN:]


def kernel(inputs, table, W, b):
    batch, seq = inputs.shape
    B = batch * seq
    info = plsc.get_sparse_core_info()
    n_workers = info.num_cores * info.num_subcores
    b_per_w = B // n_workers
    n_chunks = b_per_w // (2 * PAIR)

    flat = inputs.reshape(B)
    ia = flat[:B // 2].reshape(n_workers, n_chunks, PAIR)
    ib = flat[B // 2:].reshape(n_workers, n_chunks, PAIR)
    idx2 = jnp.concatenate([ia, ib], axis=-1)

    tableT = table.T
    table_rm = pl.pallas_call(
        _t0_body,
        grid=((N_TABLE + T0_BLK - 1) // T0_BLK,),
        in_specs=[pl.BlockSpec((HIDDEN, T0_BLK), lambda i: (0, i))],
        out_specs=pl.BlockSpec((T0_BLK, HIDDEN), lambda i: (i, 0)),
        out_shape=jax.ShapeDtypeStruct((N_TABLE, HIDDEN), jnp.float32),
    )(tableT)

    g128 = _make_gather(B, n_workers, b_per_w, n_chunks)(idx2, table_rm)

    wt = W.T
    z = jnp.zeros_like(wt)
    wd = jnp.block([[wt, z], [z, wt]])
    bcat = jnp.concatenate([b, b]).reshape(1, 2 * HIDDEN)

    out = pl.pallas_call(
        _tcf_body,
        grid=(B // (2 * TCF_BLK),),
        in_specs=[
            pl.BlockSpec((TCF_BLK, 2 * HIDDEN), lambda i: (i, 0)),
            pl.BlockSpec((2 * HIDDEN, 2 * HIDDEN), lambda i: (0, 0)),
            pl.BlockSpec((1, 2 * HIDDEN), lambda i: (0, 0)),
        ],
        out_specs=pl.BlockSpec((2, TCF_BLK, HIDDEN), lambda i: (0, i, 0)),
        out_shape=jax.ShapeDtypeStruct((2, B // 2, HIDDEN), jnp.float32),
    )(g128, wd, bcat)

    return out.reshape(batch, seq, HIDDEN)
